# baseline (speedup 1.0000x reference)
import dataclasses
import functools

import jax
import jax.numpy as jnp
from jax import lax
from jax.experimental import pallas as pl
from jax.experimental.pallas import tpu as pltpu
from jax.experimental.pallas import tpu_sc as plsc

N = 10000
E = 320000
D = 128
G = 64

NPAD = 10240
NC = 2
NS = 16
NW = NC * NS
C = 128
CH = (E + NW * C - 1) // (NW * C)
CH += CH % 2
EP = NW * CH * C
ZD = NPAD // NS
ZR = 64
BR = 1024
NBLK = NPAD // BR

_sc_mesh = functools.partial(
    plsc.VectorSubcoreMesh, core_axis_name="c", subcore_axis_name="s")

_sc_params = pltpu.CompilerParams()
if "needs_layout_passes" in pltpu.CompilerParams.__dataclass_fields__:
  _sc_params = dataclasses.replace(_sc_params, needs_layout_passes=False)


def _deg_partials(edata, zdeg):
  @functools.partial(
      pl.kernel,
      out_type=jax.ShapeDtypeStruct((NC, NPAD), jnp.float32),
      mesh=_sc_mesh(),
      compiler_params=_sc_params,
      scratch_types=[
          pltpu.VMEM((CH, 3, C), jnp.int32),
          pltpu.VMEM_SHARED((NPAD,), jnp.float32),
      ],
  )
  def k(ed_hbm, z_hbm, out_hbm, ed_v, deg_s):
    core = lax.axis_index("c")
    sub = lax.axis_index("s")
    w = core * NS + sub
    pltpu.sync_copy(z_hbm.at[pl.ds(sub * ZD, ZD)],
                    deg_s.at[pl.ds(sub * ZD, ZD)])
    pltpu.sync_copy(ed_hbm.at[w], ed_v)
    plsc.subcore_barrier()

    @pl.loop(0, CH)
    def _(j):
      pltpu.sync_copy(ed_v.bitcast(jnp.float32).at[j, 2],
                      deg_s.at[ed_v.at[j, 1]], add=True)

    plsc.subcore_barrier()

    @pl.when(sub == 0)
    def _():
      pltpu.sync_copy(deg_s, out_hbm.at[core])

  return k(edata, zdeg)


def _agg_partials(y, edata, zrow):
  @functools.partial(
      pl.kernel,
      out_type=jax.ShapeDtypeStruct((NC, NPAD, D), jnp.float32),
      mesh=_sc_mesh(),
      compiler_params=_sc_params,
      scratch_types=[
          pltpu.VMEM((3, C), jnp.int32),
          pltpu.VMEM((3, C), jnp.int32),
          pltpu.VMEM((2, C), jnp.int32),
          pltpu.VMEM((2, C), jnp.int32),
          pltpu.VMEM((C, D), jnp.float32),
          pltpu.VMEM((C, D), jnp.float32),
          pltpu.VMEM_SHARED((NPAD, D), jnp.float32),
          pltpu.SemaphoreType.DMA,
          pltpu.SemaphoreType.DMA,
          pltpu.SemaphoreType.DMA,
          pltpu.SemaphoreType.DMA,
          pltpu.SemaphoreType.DMA,
          pltpu.SemaphoreType.DMA,
      ],
  )
  def k(y_hbm, ed_hbm, z_hbm, out_hbm,
        eb0_v, eb1_v, cb0_v, cb1_v, buf0_v, buf1_v, acc_s,
        es0, es1, gs0, gs1, ss0, ss1):
    core = lax.axis_index("c")
    sub = lax.axis_index("s")
    w = core * NS + sub

    pltpu.sync_copy(z_hbm, acc_s.at[pl.ds(sub * ZD, ZD), :])

    pltpu.async_copy(ed_hbm.at[w, 0], eb0_v, es0)
    pltpu.async_copy(ed_hbm.at[w, 1], eb1_v, es1)
    pltpu.make_async_copy(ed_hbm.at[w, 0], eb0_v, es0).wait()
    pltpu.async_copy(y_hbm.at[eb0_v.at[0]], buf0_v, gs0)
    plsc.subcore_barrier()

    ebs = (eb0_v, eb1_v)
    cbs = (cb0_v, cb1_v)
    bufs = (buf0_v, buf1_v)
    esems = (es0, es1)
    gsems = (gs0, gs1)
    ssems = (ss0, ss1)

    one = jnp.full((16,), 1, jnp.int32)

    @pl.loop(0, CH, step=2)
    def _(j):
      for b in (0, 1):
        jj = j + b
        nb = 1 - b
        eb_b, eb_n = ebs[b], ebs[nb]
        cb_b, cb_n = cbs[b], cbs[nb]
        buf_b, buf_n = bufs[b], bufs[nb]

        @pl.when(jj >= 1)
        def _():
          pltpu.make_async_copy(
              buf_n, acc_s.at[cb_n.at[0]], ssems[nb]).wait()

        @pl.when(jj + 1 < CH)
        def _():
          pltpu.make_async_copy(ed_hbm.at[w, jj + 1], eb_n, esems[nb]).wait()
          pltpu.async_copy(y_hbm.at[eb_n.at[0]], buf_n, gsems[nb])

        pltpu.make_async_copy(y_hbm.at[eb_b.at[0]], buf_b, gsems[b]).wait()

        for f in range(D // 16):
          sl = pl.ds(f * 16, 16)
          cb_b[0, sl] = eb_b[1, sl]
          cb_b[1, sl] = eb_b[2, sl]

        @pl.when(jj + 2 < CH)
        def _():
          pltpu.async_copy(ed_hbm.at[w, jj + 2], eb_b, esems[b])

        @plsc.parallel_loop(0, C, unroll=8)
        def _(kk):
          wv = plsc.bitcast(
              plsc.load_gather(cb_b, [one, jnp.full((16,), kk, jnp.int32)]),
              jnp.float32)
          for f in range(D // 16):
            sl = pl.ds(f * 16, 16)
            buf_b[kk, sl] = buf_b[kk, sl] * wv

        pltpu.async_copy(buf_b, acc_s.at[cb_b.at[0]], ssems[b], add=True)

    pltpu.make_async_copy(buf1_v, acc_s.at[cb1_v.at[0]], ss1).wait()
    plsc.subcore_barrier()

    @pl.when(sub == 0)
    def _():
      pltpu.sync_copy(acc_s, out_hbm.at[core])

  return k(y, edata, zrow)


def _dinv_of(dp_ref):
  deg = dp_ref[0] + dp_ref[1] + 1.0
  return lax.rsqrt(deg)


def _mm_scale(xp, W, degp3):
  def body(x_ref, w_ref, dp_ref, o_ref):
    dinv = _dinv_of(dp_ref)
    mm = lax.dot_general(x_ref[...].astype(jnp.bfloat16),
                         w_ref[...].astype(jnp.bfloat16),
                         (((1,), (1,)), ((), ())),
                         preferred_element_type=jnp.float32)
    o_ref[...] = mm * dinv

  return pl.pallas_call(
      body,
      grid=(NBLK,),
      in_specs=[
          pl.BlockSpec((BR, D), lambda i: (i, 0)),
          pl.BlockSpec((D, D), lambda i: (0, 0)),
          pl.BlockSpec((2, BR, 1), lambda i: (0, i, 0)),
      ],
      out_specs=pl.BlockSpec((BR, D), lambda i: (i, 0)),
      out_shape=jax.ShapeDtypeStruct((NPAD, D), jnp.float32),
  )(xp, W, degp3)


def _combine_mm(aggp, y, degp3, b2d, W):
  def body(a_ref, y_ref, dp_ref, b_ref, w_ref, o_ref):
    dinv = _dinv_of(dp_ref)
    agg = a_ref[0] + a_ref[1]
    h = jnp.maximum(dinv * (agg + y_ref[...]) + b_ref[...], 0.0)
    mm = lax.dot_general(h.astype(jnp.bfloat16),
                         w_ref[...].astype(jnp.bfloat16),
                         (((1,), (1,)), ((), ())),
                         preferred_element_type=jnp.float32)
    o_ref[...] = mm * dinv

  return pl.pallas_call(
      body,
      grid=(NBLK,),
      in_specs=[
          pl.BlockSpec((2, BR, D), lambda i: (0, i, 0)),
          pl.BlockSpec((BR, D), lambda i: (i, 0)),
          pl.BlockSpec((2, BR, 1), lambda i: (0, i, 0)),
          pl.BlockSpec((1, D), lambda i: (0, 0)),
          pl.BlockSpec((D, D), lambda i: (0, 0)),
      ],
      out_specs=pl.BlockSpec((BR, D), lambda i: (i, 0)),
      out_shape=jax.ShapeDtypeStruct((NPAD, D), jnp.float32),
  )(aggp, y, degp3, b2d, W)


def _final(aggp, y, degp3, b2d, batch3, Wfc, bfc2d):
  def body(a_ref, y_ref, dp_ref, b_ref, bat_ref, wfc_ref, bfc_ref, o_ref,
           ps_ref, cnt_ref):
    i = pl.program_id(0)

    @pl.when(i == 0)
    def _():
      ps_ref[...] = jnp.zeros_like(ps_ref)
      cnt_ref[...] = jnp.zeros_like(cnt_ref)

    dinv = _dinv_of(dp_ref)
    agg = a_ref[0] + a_ref[1]
    h = jnp.maximum(dinv * (agg + y_ref[...]) + b_ref[...], 0.0)
    bat = bat_ref[0]
    oh = (bat == lax.broadcasted_iota(jnp.int32, (1, G), 1)
          ).astype(jnp.float32)
    ps_ref[...] += lax.dot_general(oh, h, (((0,), (0,)), ((), ())),
                                   preferred_element_type=jnp.float32)
    ones = jnp.ones((BR, 1), jnp.float32)
    cnt_ref[...] += lax.dot_general(oh, ones, (((0,), (0,)), ((), ())),
                                    preferred_element_type=jnp.float32)

    @pl.when(i == pl.num_programs(0) - 1)
    def _():
      p = ps_ref[...] / jnp.maximum(cnt_ref[...], 1.0)
      o_ref[...] = lax.dot_general(p, wfc_ref[...], (((1,), (1,)), ((), ())),
                                   preferred_element_type=jnp.float32
                                   ) + bfc_ref[...]

  return pl.pallas_call(
      body,
      grid=(NBLK,),
      in_specs=[
          pl.BlockSpec((2, BR, D), lambda i: (0, i, 0)),
          pl.BlockSpec((BR, D), lambda i: (i, 0)),
          pl.BlockSpec((2, BR, 1), lambda i: (0, i, 0)),
          pl.BlockSpec((1, D), lambda i: (0, 0)),
          pl.BlockSpec((1, BR, 1), lambda i: (i, 0, 0)),
          pl.BlockSpec((D, D), lambda i: (0, 0)),
          pl.BlockSpec((1, D), lambda i: (0, 0)),
      ],
      out_specs=pl.BlockSpec((G, D), lambda i: (0, 0)),
      out_shape=jax.ShapeDtypeStruct((G, D), jnp.float32),
      scratch_shapes=[
          pltpu.VMEM((G, D), jnp.float32),
          pltpu.VMEM((G, 1), jnp.float32),
      ],
  )(aggp, y, degp3, b2d, batch3, Wfc, bfc2d)


@jax.jit
def kernel(x, edge_index, edge_weight, batch, W1, b1, W2, b2, Wfc, bfc):
  xp = jnp.zeros((NPAD, D), jnp.float32).at[:N].set(x)

  pad = EP - E
  padidx = (jnp.arange(pad, dtype=jnp.int32) * 131) % N
  row_p = jnp.concatenate([edge_index[0], padidx]).reshape(NW, CH, C)
  col_p = jnp.concatenate([edge_index[1], padidx]).reshape(NW, CH, C)
  ew_b = lax.bitcast_convert_type(
      jnp.concatenate([edge_weight, jnp.zeros((pad,), jnp.float32)]),
      jnp.int32).reshape(NW, CH, C)
  edata = jnp.stack([row_p, col_p, ew_b], axis=2)

  batch3 = jnp.concatenate(
      [batch, jnp.full((NPAD - N,), G, jnp.int32)]).reshape(NBLK, BR, 1)

  zdeg = jnp.zeros((NPAD,), jnp.float32)
  zrow = jnp.zeros((ZD, D), jnp.float32)

  degp3 = _deg_partials(edata, zdeg).reshape(NC, NPAD, 1)

  y1 = _mm_scale(xp, W1, degp3)
  agg1 = _agg_partials(y1, edata, zrow)
  y2 = _combine_mm(agg1, y1, degp3, b1.reshape(1, D), W2)
  agg2 = _agg_partials(y2, edata, zrow)
  return _final(agg2, y2, degp3, b2.reshape(1, D), batch3, Wfc,
                bfc.reshape(1, D))


# speedup vs baseline: 1.0004x; 1.0004x over previous
import dataclasses
import functools

import jax
import jax.numpy as jnp
from jax import lax
from jax.experimental import pallas as pl
from jax.experimental.pallas import tpu as pltpu
from jax.experimental.pallas import tpu_sc as plsc

N = 10000
E = 320000
D = 128
G = 64

NPAD = 10240
NC = 2
NS = 16
NW = NC * NS
C = 128
CH = (E + NW * C - 1) // (NW * C)
CH += CH % 2
EP = NW * CH * C
ZD = NPAD // NS
ZR = 64
BR = 1024
NBLK = NPAD // BR

_sc_mesh = functools.partial(
    plsc.VectorSubcoreMesh, core_axis_name="c", subcore_axis_name="s")

_sc_params = pltpu.CompilerParams()
if "needs_layout_passes" in pltpu.CompilerParams.__dataclass_fields__:
  _sc_params = dataclasses.replace(_sc_params, needs_layout_passes=False)


def _deg_partials(edata, zdeg):
  @functools.partial(
      pl.kernel,
      out_type=jax.ShapeDtypeStruct((NC, NPAD), jnp.float32),
      mesh=_sc_mesh(),
      compiler_params=_sc_params,
      scratch_types=[
          pltpu.VMEM((CH, 3, C), jnp.int32),
          pltpu.VMEM_SHARED((NPAD,), jnp.float32),
      ],
  )
  def k(ed_hbm, z_hbm, out_hbm, ed_v, deg_s):
    core = lax.axis_index("c")
    sub = lax.axis_index("s")
    w = core * NS + sub
    pltpu.sync_copy(z_hbm.at[pl.ds(sub * ZD, ZD)],
                    deg_s.at[pl.ds(sub * ZD, ZD)])
    pltpu.sync_copy(ed_hbm.at[w], ed_v)
    plsc.subcore_barrier()

    @pl.loop(0, CH)
    def _(j):
      pltpu.sync_copy(ed_v.bitcast(jnp.float32).at[j, 2],
                      deg_s.at[ed_v.at[j, 1]], add=True)

    plsc.subcore_barrier()
    pltpu.sync_copy(deg_s.at[pl.ds(sub * ZD, ZD)],
                    out_hbm.at[core, pl.ds(sub * ZD, ZD)])

  return k(edata, zdeg)


def _agg_partials(y, edata, zrow):
  @functools.partial(
      pl.kernel,
      out_type=jax.ShapeDtypeStruct((NC, NPAD, D), jnp.float32),
      mesh=_sc_mesh(),
      compiler_params=_sc_params,
      scratch_types=[
          pltpu.VMEM((3, C), jnp.int32),
          pltpu.VMEM((3, C), jnp.int32),
          pltpu.VMEM((2, C), jnp.int32),
          pltpu.VMEM((2, C), jnp.int32),
          pltpu.VMEM((C, D), jnp.float32),
          pltpu.VMEM((C, D), jnp.float32),
          pltpu.VMEM_SHARED((NPAD, D), jnp.float32),
          pltpu.SemaphoreType.DMA,
          pltpu.SemaphoreType.DMA,
          pltpu.SemaphoreType.DMA,
          pltpu.SemaphoreType.DMA,
          pltpu.SemaphoreType.DMA,
          pltpu.SemaphoreType.DMA,
      ],
  )
  def k(y_hbm, ed_hbm, z_hbm, out_hbm,
        eb0_v, eb1_v, cb0_v, cb1_v, buf0_v, buf1_v, acc_s,
        es0, es1, gs0, gs1, ss0, ss1):
    core = lax.axis_index("c")
    sub = lax.axis_index("s")
    w = core * NS + sub

    pltpu.sync_copy(z_hbm, acc_s.at[pl.ds(sub * ZD, ZD), :])

    pltpu.async_copy(ed_hbm.at[w, 0], eb0_v, es0)
    pltpu.async_copy(ed_hbm.at[w, 1], eb1_v, es1)
    pltpu.make_async_copy(ed_hbm.at[w, 0], eb0_v, es0).wait()
    pltpu.async_copy(y_hbm.at[eb0_v.at[0]], buf0_v, gs0)
    plsc.subcore_barrier()

    ebs = (eb0_v, eb1_v)
    cbs = (cb0_v, cb1_v)
    bufs = (buf0_v, buf1_v)
    esems = (es0, es1)
    gsems = (gs0, gs1)
    ssems = (ss0, ss1)

    one = jnp.full((16,), 1, jnp.int32)

    @pl.loop(0, CH, step=2)
    def _(j):
      for b in (0, 1):
        jj = j + b
        nb = 1 - b
        eb_b, eb_n = ebs[b], ebs[nb]
        cb_b, cb_n = cbs[b], cbs[nb]
        buf_b, buf_n = bufs[b], bufs[nb]

        @pl.when(jj >= 1)
        def _():
          pltpu.make_async_copy(
              buf_n, acc_s.at[cb_n.at[0]], ssems[nb]).wait()

        @pl.when(jj + 1 < CH)
        def _():
          pltpu.make_async_copy(ed_hbm.at[w, jj + 1], eb_n, esems[nb]).wait()
          pltpu.async_copy(y_hbm.at[eb_n.at[0]], buf_n, gsems[nb])

        pltpu.make_async_copy(y_hbm.at[eb_b.at[0]], buf_b, gsems[b]).wait()

        for f in range(D // 16):
          sl = pl.ds(f * 16, 16)
          cb_b[0, sl] = eb_b[1, sl]
          cb_b[1, sl] = eb_b[2, sl]

        @pl.when(jj + 2 < CH)
        def _():
          pltpu.async_copy(ed_hbm.at[w, jj + 2], eb_b, esems[b])

        @plsc.parallel_loop(0, C, unroll=8)
        def _(kk):
          wv = plsc.bitcast(
              plsc.load_gather(cb_b, [one, jnp.full((16,), kk, jnp.int32)]),
              jnp.float32)
          for f in range(D // 16):
            sl = pl.ds(f * 16, 16)
            buf_b[kk, sl] = buf_b[kk, sl] * wv

        pltpu.async_copy(buf_b, acc_s.at[cb_b.at[0]], ssems[b], add=True)

    pltpu.make_async_copy(buf1_v, acc_s.at[cb1_v.at[0]], ss1).wait()
    plsc.subcore_barrier()
    pltpu.sync_copy(acc_s.at[pl.ds(sub * ZD, ZD), :],
                    out_hbm.at[core, pl.ds(sub * ZD, ZD), :])

  return k(y, edata, zrow)


def _dinv_of(dp_ref):
  deg = dp_ref[0] + dp_ref[1] + 1.0
  return lax.rsqrt(deg)


def _mm_scale(xp, W, degp3):
  def body(x_ref, w_ref, dp_ref, o_ref):
    dinv = _dinv_of(dp_ref)
    mm = lax.dot_general(x_ref[...].astype(jnp.bfloat16),
                         w_ref[...].astype(jnp.bfloat16),
                         (((1,), (1,)), ((), ())),
                         preferred_element_type=jnp.float32)
    o_ref[...] = mm * dinv

  return pl.pallas_call(
      body,
      grid=(NBLK,),
      in_specs=[
          pl.BlockSpec((BR, D), lambda i: (i, 0)),
          pl.BlockSpec((D, D), lambda i: (0, 0)),
          pl.BlockSpec((2, BR, 1), lambda i: (0, i, 0)),
      ],
      out_specs=pl.BlockSpec((BR, D), lambda i: (i, 0)),
      out_shape=jax.ShapeDtypeStruct((NPAD, D), jnp.float32),
  )(xp, W, degp3)


def _combine_mm(aggp, y, degp3, b2d, W):
  def body(a_ref, y_ref, dp_ref, b_ref, w_ref, o_ref):
    dinv = _dinv_of(dp_ref)
    agg = a_ref[0] + a_ref[1]
    h = jnp.maximum(dinv * (agg + y_ref[...]) + b_ref[...], 0.0)
    mm = lax.dot_general(h.astype(jnp.bfloat16),
                         w_ref[...].astype(jnp.bfloat16),
                         (((1,), (1,)), ((), ())),
                         preferred_element_type=jnp.float32)
    o_ref[...] = mm * dinv

  return pl.pallas_call(
      body,
      grid=(NBLK,),
      in_specs=[
          pl.BlockSpec((2, BR, D), lambda i: (0, i, 0)),
          pl.BlockSpec((BR, D), lambda i: (i, 0)),
          pl.BlockSpec((2, BR, 1), lambda i: (0, i, 0)),
          pl.BlockSpec((1, D), lambda i: (0, 0)),
          pl.BlockSpec((D, D), lambda i: (0, 0)),
      ],
      out_specs=pl.BlockSpec((BR, D), lambda i: (i, 0)),
      out_shape=jax.ShapeDtypeStruct((NPAD, D), jnp.float32),
  )(aggp, y, degp3, b2d, W)


def _final(aggp, y, degp3, b2d, batch3, Wfc, bfc2d):
  def body(a_ref, y_ref, dp_ref, b_ref, bat_ref, wfc_ref, bfc_ref, o_ref,
           ps_ref, cnt_ref):
    i = pl.program_id(0)

    @pl.when(i == 0)
    def _():
      ps_ref[...] = jnp.zeros_like(ps_ref)
      cnt_ref[...] = jnp.zeros_like(cnt_ref)

    dinv = _dinv_of(dp_ref)
    agg = a_ref[0] + a_ref[1]
    h = jnp.maximum(dinv * (agg + y_ref[...]) + b_ref[...], 0.0)
    bat = bat_ref[0]
    oh = (bat == lax.broadcasted_iota(jnp.int32, (1, G), 1)
          ).astype(jnp.float32)
    ps_ref[...] += lax.dot_general(oh, h, (((0,), (0,)), ((), ())),
                                   preferred_element_type=jnp.float32)
    ones = jnp.ones((BR, 1), jnp.float32)
    cnt_ref[...] += lax.dot_general(oh, ones, (((0,), (0,)), ((), ())),
                                    preferred_element_type=jnp.float32)

    @pl.when(i == pl.num_programs(0) - 1)
    def _():
      p = ps_ref[...] / jnp.maximum(cnt_ref[...], 1.0)
      o_ref[...] = lax.dot_general(p, wfc_ref[...], (((1,), (1,)), ((), ())),
                                   preferred_element_type=jnp.float32
                                   ) + bfc_ref[...]

  return pl.pallas_call(
      body,
      grid=(NBLK,),
      in_specs=[
          pl.BlockSpec((2, BR, D), lambda i: (0, i, 0)),
          pl.BlockSpec((BR, D), lambda i: (i, 0)),
          pl.BlockSpec((2, BR, 1), lambda i: (0, i, 0)),
          pl.BlockSpec((1, D), lambda i: (0, 0)),
          pl.BlockSpec((1, BR, 1), lambda i: (i, 0, 0)),
          pl.BlockSpec((D, D), lambda i: (0, 0)),
          pl.BlockSpec((1, D), lambda i: (0, 0)),
      ],
      out_specs=pl.BlockSpec((G, D), lambda i: (0, 0)),
      out_shape=jax.ShapeDtypeStruct((G, D), jnp.float32),
      scratch_shapes=[
          pltpu.VMEM((G, D), jnp.float32),
          pltpu.VMEM((G, 1), jnp.float32),
      ],
  )(aggp, y, degp3, b2d, batch3, Wfc, bfc2d)


@jax.jit
def kernel(x, edge_index, edge_weight, batch, W1, b1, W2, b2, Wfc, bfc):
  xp = jnp.zeros((NPAD, D), jnp.float32).at[:N].set(x)

  pad = EP - E
  padidx = (jnp.arange(pad, dtype=jnp.int32) * 131) % N
  row_p = jnp.concatenate([edge_index[0], padidx]).reshape(NW, CH, C)
  col_p = jnp.concatenate([edge_index[1], padidx]).reshape(NW, CH, C)
  ew_b = lax.bitcast_convert_type(
      jnp.concatenate([edge_weight, jnp.zeros((pad,), jnp.float32)]),
      jnp.int32).reshape(NW, CH, C)
  edata = jnp.stack([row_p, col_p, ew_b], axis=2)

  batch3 = jnp.concatenate(
      [batch, jnp.full((NPAD - N,), G, jnp.int32)]).reshape(NBLK, BR, 1)

  zdeg = jnp.zeros((NPAD,), jnp.float32)
  zrow = jnp.zeros((ZD, D), jnp.float32)

  degp3 = _deg_partials(edata, zdeg).reshape(NC, NPAD, 1)

  y1 = _mm_scale(xp, W1, degp3)
  agg1 = _agg_partials(y1, edata, zrow)
  y2 = _combine_mm(agg1, y1, degp3, b1.reshape(1, D), W2)
  agg2 = _agg_partials(y2, edata, zrow)
  return _final(agg2, y2, degp3, b2.reshape(1, D), batch3, Wfc,
                bfc.reshape(1, D))


# speedup vs baseline: 1.0026x; 1.0022x over previous
import dataclasses
import functools

import jax
import jax.numpy as jnp
from jax import lax
from jax.experimental import pallas as pl
from jax.experimental.pallas import tpu as pltpu
from jax.experimental.pallas import tpu_sc as plsc

N = 10000
E = 320000
D = 128
G = 64

NPAD = 10240
NC = 2
NS = 16
NW = NC * NS
C = 128
CH = (E + NW * C - 1) // (NW * C)
CH += CH % 2
EP = NW * CH * C
ZD = NPAD // NS
ZR = 64
BR = 1024
NBLK = NPAD // BR

_sc_mesh = functools.partial(
    plsc.VectorSubcoreMesh, core_axis_name="c", subcore_axis_name="s")

_sc_params = pltpu.CompilerParams()
if "needs_layout_passes" in pltpu.CompilerParams.__dataclass_fields__:
  _sc_params = dataclasses.replace(_sc_params, needs_layout_passes=False)


def _deg_partials(cwdata, zdeg):
  @functools.partial(
      pl.kernel,
      out_type=jax.ShapeDtypeStruct((NC, NPAD), jnp.float32),
      mesh=_sc_mesh(),
      compiler_params=_sc_params,
      scratch_types=[
          pltpu.VMEM((CH, 2, C), jnp.int32),
          pltpu.VMEM_SHARED((NPAD,), jnp.float32),
      ],
  )
  def k(cw_hbm, z_hbm, out_hbm, cw_v, deg_s):
    core = lax.axis_index("c")
    sub = lax.axis_index("s")
    w = core * NS + sub
    pltpu.sync_copy(z_hbm.at[pl.ds(sub * ZD, ZD)],
                    deg_s.at[pl.ds(sub * ZD, ZD)])
    pltpu.sync_copy(cw_hbm.at[w], cw_v)
    plsc.subcore_barrier()

    @pl.loop(0, CH)
    def _(j):
      pltpu.sync_copy(cw_v.bitcast(jnp.float32).at[j, 1],
                      deg_s.at[cw_v.at[j, 0]], add=True)

    plsc.subcore_barrier()
    pltpu.sync_copy(deg_s.at[pl.ds(sub * ZD, ZD)],
                    out_hbm.at[core, pl.ds(sub * ZD, ZD)])

  return k(cwdata, zdeg)


def _agg_partials(y, edata, zrow):
  @functools.partial(
      pl.kernel,
      out_type=jax.ShapeDtypeStruct((NC, NPAD, D), jnp.float32),
      mesh=_sc_mesh(),
      compiler_params=_sc_params,
      scratch_types=[
          pltpu.VMEM((3, C), jnp.int32),
          pltpu.VMEM((3, C), jnp.int32),
          pltpu.VMEM((2, C), jnp.int32),
          pltpu.VMEM((2, C), jnp.int32),
          pltpu.VMEM((C, D), jnp.float32),
          pltpu.VMEM((C, D), jnp.float32),
          pltpu.VMEM_SHARED((NPAD, D), jnp.float32),
          pltpu.SemaphoreType.DMA,
          pltpu.SemaphoreType.DMA,
          pltpu.SemaphoreType.DMA,
          pltpu.SemaphoreType.DMA,
          pltpu.SemaphoreType.DMA,
          pltpu.SemaphoreType.DMA,
      ],
  )
  def k(y_hbm, ed_hbm, z_hbm, out_hbm,
        eb0_v, eb1_v, cb0_v, cb1_v, buf0_v, buf1_v, acc_s,
        es0, es1, gs0, gs1, ss0, ss1):
    core = lax.axis_index("c")
    sub = lax.axis_index("s")
    w = core * NS + sub

    pltpu.sync_copy(z_hbm, acc_s.at[pl.ds(sub * ZD, ZD), :])

    pltpu.async_copy(ed_hbm.at[w, 0], eb0_v, es0)
    pltpu.async_copy(ed_hbm.at[w, 1], eb1_v, es1)
    pltpu.make_async_copy(ed_hbm.at[w, 0], eb0_v, es0).wait()
    pltpu.async_copy(y_hbm.at[eb0_v.at[0]], buf0_v, gs0)
    plsc.subcore_barrier()

    ebs = (eb0_v, eb1_v)
    cbs = (cb0_v, cb1_v)
    bufs = (buf0_v, buf1_v)
    esems = (es0, es1)
    gsems = (gs0, gs1)
    ssems = (ss0, ss1)

    one = jnp.full((16,), 1, jnp.int32)

    @pl.loop(0, CH, step=2)
    def _(j):
      for b in (0, 1):
        jj = j + b
        nb = 1 - b
        eb_b, eb_n = ebs[b], ebs[nb]
        cb_b, cb_n = cbs[b], cbs[nb]
        buf_b, buf_n = bufs[b], bufs[nb]

        @pl.when(jj >= 1)
        def _():
          pltpu.make_async_copy(
              buf_n, acc_s.at[cb_n.at[0]], ssems[nb]).wait()

        @pl.when(jj + 1 < CH)
        def _():
          pltpu.make_async_copy(ed_hbm.at[w, jj + 1], eb_n, esems[nb]).wait()
          pltpu.async_copy(y_hbm.at[eb_n.at[0]], buf_n, gsems[nb])

        pltpu.make_async_copy(y_hbm.at[eb_b.at[0]], buf_b, gsems[b]).wait()

        for f in range(D // 16):
          sl = pl.ds(f * 16, 16)
          cb_b[0, sl] = eb_b[1, sl]
          cb_b[1, sl] = eb_b[2, sl]

        @pl.when(jj + 2 < CH)
        def _():
          pltpu.async_copy(ed_hbm.at[w, jj + 2], eb_b, esems[b])

        @plsc.parallel_loop(0, C, unroll=16)
        def _(kk):
          wv = plsc.bitcast(
              plsc.load_gather(cb_b, [one, jnp.full((16,), kk, jnp.int32)]),
              jnp.float32)
          for f in range(D // 16):
            sl = pl.ds(f * 16, 16)
            buf_b[kk, sl] = buf_b[kk, sl] * wv

        pltpu.async_copy(buf_b, acc_s.at[cb_b.at[0]], ssems[b], add=True)

    pltpu.make_async_copy(buf1_v, acc_s.at[cb1_v.at[0]], ss1).wait()
    plsc.subcore_barrier()
    pltpu.sync_copy(acc_s.at[pl.ds(sub * ZD, ZD), :],
                    out_hbm.at[core, pl.ds(sub * ZD, ZD), :])

  return k(y, edata, zrow)


def _dinv_of(dp_ref):
  deg = dp_ref[0] + dp_ref[1] + 1.0
  return lax.rsqrt(deg)


def _mm_scale(xp, W, degp3):
  def body(x_ref, w_ref, dp_ref, o_ref):
    dinv = _dinv_of(dp_ref)
    mm = lax.dot_general(x_ref[...].astype(jnp.bfloat16),
                         w_ref[...].astype(jnp.bfloat16),
                         (((1,), (1,)), ((), ())),
                         preferred_element_type=jnp.float32)
    o_ref[...] = mm * dinv

  return pl.pallas_call(
      body,
      grid=(NBLK,),
      in_specs=[
          pl.BlockSpec((BR, D), lambda i: (i, 0)),
          pl.BlockSpec((D, D), lambda i: (0, 0)),
          pl.BlockSpec((2, BR, 1), lambda i: (0, i, 0)),
      ],
      out_specs=pl.BlockSpec((BR, D), lambda i: (i, 0)),
      out_shape=jax.ShapeDtypeStruct((NPAD, D), jnp.float32),
  )(xp, W, degp3)


def _combine_mm(aggp, y, degp3, b2d, W):
  def body(a_ref, y_ref, dp_ref, b_ref, w_ref, o_ref):
    dinv = _dinv_of(dp_ref)
    agg = a_ref[0] + a_ref[1]
    h = jnp.maximum(dinv * (agg + y_ref[...]) + b_ref[...], 0.0)
    mm = lax.dot_general(h.astype(jnp.bfloat16),
                         w_ref[...].astype(jnp.bfloat16),
                         (((1,), (1,)), ((), ())),
                         preferred_element_type=jnp.float32)
    o_ref[...] = mm * dinv

  return pl.pallas_call(
      body,
      grid=(NBLK,),
      in_specs=[
          pl.BlockSpec((2, BR, D), lambda i: (0, i, 0)),
          pl.BlockSpec((BR, D), lambda i: (i, 0)),
          pl.BlockSpec((2, BR, 1), lambda i: (0, i, 0)),
          pl.BlockSpec((1, D), lambda i: (0, 0)),
          pl.BlockSpec((D, D), lambda i: (0, 0)),
      ],
      out_specs=pl.BlockSpec((BR, D), lambda i: (i, 0)),
      out_shape=jax.ShapeDtypeStruct((NPAD, D), jnp.float32),
  )(aggp, y, degp3, b2d, W)


def _final(aggp, y, degp3, b2d, batch3, Wfc, bfc2d):
  def body(a_ref, y_ref, dp_ref, b_ref, bat_ref, wfc_ref, bfc_ref, o_ref,
           ps_ref, cnt_ref):
    i = pl.program_id(0)

    @pl.when(i == 0)
    def _():
      ps_ref[...] = jnp.zeros_like(ps_ref)
      cnt_ref[...] = jnp.zeros_like(cnt_ref)

    dinv = _dinv_of(dp_ref)
    agg = a_ref[0] + a_ref[1]
    h = jnp.maximum(dinv * (agg + y_ref[...]) + b_ref[...], 0.0)
    bat = bat_ref[0]
    oh = (bat == lax.broadcasted_iota(jnp.int32, (1, G), 1)
          ).astype(jnp.float32)
    ps_ref[...] += lax.dot_general(oh, h, (((0,), (0,)), ((), ())),
                                   preferred_element_type=jnp.float32)
    ones = jnp.ones((BR, 1), jnp.float32)
    cnt_ref[...] += lax.dot_general(oh, ones, (((0,), (0,)), ((), ())),
                                    preferred_element_type=jnp.float32)

    @pl.when(i == pl.num_programs(0) - 1)
    def _():
      p = ps_ref[...] / jnp.maximum(cnt_ref[...], 1.0)
      o_ref[...] = lax.dot_general(p, wfc_ref[...], (((1,), (1,)), ((), ())),
                                   preferred_element_type=jnp.float32
                                   ) + bfc_ref[...]

  return pl.pallas_call(
      body,
      grid=(NBLK,),
      in_specs=[
          pl.BlockSpec((2, BR, D), lambda i: (0, i, 0)),
          pl.BlockSpec((BR, D), lambda i: (i, 0)),
          pl.BlockSpec((2, BR, 1), lambda i: (0, i, 0)),
          pl.BlockSpec((1, D), lambda i: (0, 0)),
          pl.BlockSpec((1, BR, 1), lambda i: (i, 0, 0)),
          pl.BlockSpec((D, D), lambda i: (0, 0)),
          pl.BlockSpec((1, D), lambda i: (0, 0)),
      ],
      out_specs=pl.BlockSpec((G, D), lambda i: (0, 0)),
      out_shape=jax.ShapeDtypeStruct((G, D), jnp.float32),
      scratch_shapes=[
          pltpu.VMEM((G, D), jnp.float32),
          pltpu.VMEM((G, 1), jnp.float32),
      ],
  )(aggp, y, degp3, b2d, batch3, Wfc, bfc2d)


@jax.jit
def kernel(x, edge_index, edge_weight, batch, W1, b1, W2, b2, Wfc, bfc):
  xp = jnp.zeros((NPAD, D), jnp.float32).at[:N].set(x)

  pad = EP - E
  padidx = (jnp.arange(pad, dtype=jnp.int32) * 131) % N
  row_p = jnp.concatenate([edge_index[0], padidx]).reshape(NW, CH, C)
  col_p = jnp.concatenate([edge_index[1], padidx]).reshape(NW, CH, C)
  ew_b = lax.bitcast_convert_type(
      jnp.concatenate([edge_weight, jnp.zeros((pad,), jnp.float32)]),
      jnp.int32).reshape(NW, CH, C)
  edata = jnp.stack([row_p, col_p, ew_b], axis=2)
  cwdata = jnp.stack([col_p, ew_b], axis=2)

  batch3 = jnp.concatenate(
      [batch, jnp.full((NPAD - N,), G, jnp.int32)]).reshape(NBLK, BR, 1)

  zdeg = jnp.zeros((NPAD,), jnp.float32)
  zrow = jnp.zeros((ZD, D), jnp.float32)

  degp3 = _deg_partials(cwdata, zdeg).reshape(NC, NPAD, 1)

  y1 = _mm_scale(xp, W1, degp3)
  agg1 = _agg_partials(y1, edata, zrow)
  y2 = _combine_mm(agg1, y1, degp3, b1.reshape(1, D), W2)
  agg2 = _agg_partials(y2, edata, zrow)
  return _final(agg2, y2, degp3, b2.reshape(1, D), batch3, Wfc,
                bfc.reshape(1, D))


# speedup vs baseline: 1.0269x; 1.0243x over previous
import dataclasses
import functools

import jax
import jax.numpy as jnp
from jax import lax
from jax.experimental import pallas as pl
from jax.experimental.pallas import tpu as pltpu
from jax.experimental.pallas import tpu_sc as plsc

N = 10000
E = 320000
D = 128
G = 64

NPAD = 10240
NC = 2
NS = 16
NW = NC * NS
C = 128
CH = (E + NW * C - 1) // (NW * C)
CH += CH % 2
EP = NW * CH * C
ZD = NPAD // NS
ZR = 64
BR = 1024
NBLK = NPAD // BR

_sc_mesh = functools.partial(
    plsc.VectorSubcoreMesh, core_axis_name="c", subcore_axis_name="s")

_sc_params = pltpu.CompilerParams()
if "needs_layout_passes" in pltpu.CompilerParams.__dataclass_fields__:
  _sc_params = dataclasses.replace(_sc_params, needs_layout_passes=False)


def _deg_partials(cwdata, zdeg):
  @functools.partial(
      pl.kernel,
      out_type=jax.ShapeDtypeStruct((NC, NPAD), jnp.float32),
      mesh=_sc_mesh(),
      compiler_params=_sc_params,
      scratch_types=[
          pltpu.VMEM((CH, 2, C), jnp.int32),
          pltpu.VMEM_SHARED((NPAD,), jnp.float32),
      ],
  )
  def k(cw_hbm, z_hbm, out_hbm, cw_v, deg_s):
    core = lax.axis_index("c")
    sub = lax.axis_index("s")
    w = core * NS + sub
    pltpu.sync_copy(z_hbm.at[pl.ds(sub * ZD, ZD)],
                    deg_s.at[pl.ds(sub * ZD, ZD)])
    pltpu.sync_copy(cw_hbm.at[w], cw_v)
    plsc.subcore_barrier()

    @pl.loop(0, CH)
    def _(j):
      pltpu.sync_copy(cw_v.bitcast(jnp.float32).at[j, 1],
                      deg_s.at[cw_v.at[j, 0]], add=True)

    plsc.subcore_barrier()
    pltpu.sync_copy(deg_s.at[pl.ds(sub * ZD, ZD)],
                    out_hbm.at[core, pl.ds(sub * ZD, ZD)])

  return k(cwdata, zdeg)


def _agg_partials(y, edata):
  @functools.partial(
      pl.kernel,
      out_type=jax.ShapeDtypeStruct((NC, NPAD, D), jnp.float32),
      mesh=_sc_mesh(),
      compiler_params=_sc_params,
      scratch_types=[
          pltpu.VMEM((3, C), jnp.int32),
          pltpu.VMEM((3, C), jnp.int32),
          pltpu.VMEM((2, C), jnp.int32),
          pltpu.VMEM((2, C), jnp.int32),
          pltpu.VMEM((C, D), jnp.float32),
          pltpu.VMEM((C, D), jnp.float32),
          pltpu.VMEM((ZR, D), jnp.float32),
          pltpu.VMEM_SHARED((NPAD, D), jnp.float32),
          pltpu.SemaphoreType.DMA,
          pltpu.SemaphoreType.DMA,
          pltpu.SemaphoreType.DMA,
          pltpu.SemaphoreType.DMA,
          pltpu.SemaphoreType.DMA,
          pltpu.SemaphoreType.DMA,
      ],
  )
  def k(y_hbm, ed_hbm, out_hbm,
        eb0_v, eb1_v, cb0_v, cb1_v, buf0_v, buf1_v, z_v, acc_s,
        es0, es1, gs0, gs1, ss0, ss1):
    core = lax.axis_index("c")
    sub = lax.axis_index("s")
    w = core * NS + sub

    pltpu.async_copy(ed_hbm.at[w, 0], eb0_v, es0)
    pltpu.async_copy(ed_hbm.at[w, 1], eb1_v, es1)

    @pl.loop(0, ZR)
    def _(r):
      for f in range(D // 16):
        z_v[r, pl.ds(f * 16, 16)] = jnp.zeros((16,), jnp.float32)

    for i in range(ZD // ZR):
      pltpu.sync_copy(z_v, acc_s.at[pl.ds(sub * ZD + i * ZR, ZR), :])

    pltpu.make_async_copy(ed_hbm.at[w, 0], eb0_v, es0).wait()
    pltpu.async_copy(y_hbm.at[eb0_v.at[0]], buf0_v, gs0)
    plsc.subcore_barrier()

    ebs = (eb0_v, eb1_v)
    cbs = (cb0_v, cb1_v)
    bufs = (buf0_v, buf1_v)
    esems = (es0, es1)
    gsems = (gs0, gs1)
    ssems = (ss0, ss1)

    one = jnp.full((16,), 1, jnp.int32)

    @pl.loop(0, CH, step=2)
    def _(j):
      for b in (0, 1):
        jj = j + b
        nb = 1 - b
        eb_b, eb_n = ebs[b], ebs[nb]
        cb_b, cb_n = cbs[b], cbs[nb]
        buf_b, buf_n = bufs[b], bufs[nb]

        @pl.when(jj >= 1)
        def _():
          pltpu.make_async_copy(
              buf_n, acc_s.at[cb_n.at[0]], ssems[nb]).wait()

        @pl.when(jj + 1 < CH)
        def _():
          pltpu.make_async_copy(ed_hbm.at[w, jj + 1], eb_n, esems[nb]).wait()
          pltpu.async_copy(y_hbm.at[eb_n.at[0]], buf_n, gsems[nb])

        pltpu.make_async_copy(y_hbm.at[eb_b.at[0]], buf_b, gsems[b]).wait()

        for f in range(D // 16):
          sl = pl.ds(f * 16, 16)
          cb_b[0, sl] = eb_b[1, sl]
          cb_b[1, sl] = eb_b[2, sl]

        @pl.when(jj + 2 < CH)
        def _():
          pltpu.async_copy(ed_hbm.at[w, jj + 2], eb_b, esems[b])

        @plsc.parallel_loop(0, C, unroll=16)
        def _(kk):
          wv = plsc.bitcast(
              plsc.load_gather(cb_b, [one, jnp.full((16,), kk, jnp.int32)]),
              jnp.float32)
          for f in range(D // 16):
            sl = pl.ds(f * 16, 16)
            buf_b[kk, sl] = buf_b[kk, sl] * wv

        pltpu.async_copy(buf_b, acc_s.at[cb_b.at[0]], ssems[b], add=True)

    pltpu.make_async_copy(buf1_v, acc_s.at[cb1_v.at[0]], ss1).wait()
    plsc.subcore_barrier()
    pltpu.sync_copy(acc_s.at[pl.ds(sub * ZD, ZD), :],
                    out_hbm.at[core, pl.ds(sub * ZD, ZD), :])

  return k(y, edata)


def _dinv_of(dp_ref):
  deg = dp_ref[0] + dp_ref[1] + 1.0
  return lax.rsqrt(deg)


def _mm_scale(xp, W, degp3):
  def body(x_ref, w_ref, dp_ref, o_ref):
    dinv = _dinv_of(dp_ref)
    mm = lax.dot_general(x_ref[...].astype(jnp.bfloat16),
                         w_ref[...].astype(jnp.bfloat16),
                         (((1,), (1,)), ((), ())),
                         preferred_element_type=jnp.float32)
    o_ref[...] = mm * dinv

  return pl.pallas_call(
      body,
      grid=(NBLK,),
      in_specs=[
          pl.BlockSpec((BR, D), lambda i: (i, 0)),
          pl.BlockSpec((D, D), lambda i: (0, 0)),
          pl.BlockSpec((2, BR, 1), lambda i: (0, i, 0)),
      ],
      out_specs=pl.BlockSpec((BR, D), lambda i: (i, 0)),
      out_shape=jax.ShapeDtypeStruct((NPAD, D), jnp.float32),
  )(xp, W, degp3)


def _combine_mm(aggp, y, degp3, b2d, W):
  def body(a_ref, y_ref, dp_ref, b_ref, w_ref, o_ref):
    dinv = _dinv_of(dp_ref)
    agg = a_ref[0] + a_ref[1]
    h = jnp.maximum(dinv * (agg + y_ref[...]) + b_ref[...], 0.0)
    mm = lax.dot_general(h.astype(jnp.bfloat16),
                         w_ref[...].astype(jnp.bfloat16),
                         (((1,), (1,)), ((), ())),
                         preferred_element_type=jnp.float32)
    o_ref[...] = mm * dinv

  return pl.pallas_call(
      body,
      grid=(NBLK,),
      in_specs=[
          pl.BlockSpec((2, BR, D), lambda i: (0, i, 0)),
          pl.BlockSpec((BR, D), lambda i: (i, 0)),
          pl.BlockSpec((2, BR, 1), lambda i: (0, i, 0)),
          pl.BlockSpec((1, D), lambda i: (0, 0)),
          pl.BlockSpec((D, D), lambda i: (0, 0)),
      ],
      out_specs=pl.BlockSpec((BR, D), lambda i: (i, 0)),
      out_shape=jax.ShapeDtypeStruct((NPAD, D), jnp.float32),
  )(aggp, y, degp3, b2d, W)


def _final(aggp, y, degp3, b2d, batch3, Wfc, bfc2d):
  def body(a_ref, y_ref, dp_ref, b_ref, bat_ref, wfc_ref, bfc_ref, o_ref,
           ps_ref, cnt_ref):
    i = pl.program_id(0)

    @pl.when(i == 0)
    def _():
      ps_ref[...] = jnp.zeros_like(ps_ref)
      cnt_ref[...] = jnp.zeros_like(cnt_ref)

    dinv = _dinv_of(dp_ref)
    agg = a_ref[0] + a_ref[1]
    h = jnp.maximum(dinv * (agg + y_ref[...]) + b_ref[...], 0.0)
    bat = bat_ref[0]
    oh = (bat == lax.broadcasted_iota(jnp.int32, (1, G), 1)
          ).astype(jnp.float32)
    ps_ref[...] += lax.dot_general(oh, h, (((0,), (0,)), ((), ())),
                                   preferred_element_type=jnp.float32)
    ones = jnp.ones((BR, 1), jnp.float32)
    cnt_ref[...] += lax.dot_general(oh, ones, (((0,), (0,)), ((), ())),
                                    preferred_element_type=jnp.float32)

    @pl.when(i == pl.num_programs(0) - 1)
    def _():
      p = ps_ref[...] / jnp.maximum(cnt_ref[...], 1.0)
      o_ref[...] = lax.dot_general(p, wfc_ref[...], (((1,), (1,)), ((), ())),
                                   preferred_element_type=jnp.float32
                                   ) + bfc_ref[...]

  return pl.pallas_call(
      body,
      grid=(NBLK,),
      in_specs=[
          pl.BlockSpec((2, BR, D), lambda i: (0, i, 0)),
          pl.BlockSpec((BR, D), lambda i: (i, 0)),
          pl.BlockSpec((2, BR, 1), lambda i: (0, i, 0)),
          pl.BlockSpec((1, D), lambda i: (0, 0)),
          pl.BlockSpec((1, BR, 1), lambda i: (i, 0, 0)),
          pl.BlockSpec((D, D), lambda i: (0, 0)),
          pl.BlockSpec((1, D), lambda i: (0, 0)),
      ],
      out_specs=pl.BlockSpec((G, D), lambda i: (0, 0)),
      out_shape=jax.ShapeDtypeStruct((G, D), jnp.float32),
      scratch_shapes=[
          pltpu.VMEM((G, D), jnp.float32),
          pltpu.VMEM((G, 1), jnp.float32),
      ],
  )(aggp, y, degp3, b2d, batch3, Wfc, bfc2d)


@jax.jit
def kernel(x, edge_index, edge_weight, batch, W1, b1, W2, b2, Wfc, bfc):
  xp = jnp.zeros((NPAD, D), jnp.float32).at[:N].set(x)

  pad = EP - E
  padidx = (jnp.arange(pad, dtype=jnp.int32) * 131) % N
  row_p = jnp.concatenate([edge_index[0], padidx]).reshape(NW, CH, C)
  col_p = jnp.concatenate([edge_index[1], padidx]).reshape(NW, CH, C)
  ew_b = lax.bitcast_convert_type(
      jnp.concatenate([edge_weight, jnp.zeros((pad,), jnp.float32)]),
      jnp.int32).reshape(NW, CH, C)
  edata = jnp.stack([row_p, col_p, ew_b], axis=2)
  cwdata = jnp.stack([col_p, ew_b], axis=2)

  batch3 = jnp.concatenate(
      [batch, jnp.full((NPAD - N,), G, jnp.int32)]).reshape(NBLK, BR, 1)

  zdeg = jnp.zeros((NPAD,), jnp.float32)

  degp3 = _deg_partials(cwdata, zdeg).reshape(NC, NPAD, 1)

  y1 = _mm_scale(xp, W1, degp3)
  agg1 = _agg_partials(y1, edata)
  y2 = _combine_mm(agg1, y1, degp3, b1.reshape(1, D), W2)
  agg2 = _agg_partials(y2, edata)
  return _final(agg2, y2, degp3, b2.reshape(1, D), batch3, Wfc,
                bfc.reshape(1, D))


# speedup vs baseline: 1.0366x; 1.0094x over previous
import dataclasses
import functools

import jax
import jax.numpy as jnp
from jax import lax
from jax.experimental import pallas as pl
from jax.experimental.pallas import tpu as pltpu
from jax.experimental.pallas import tpu_sc as plsc

N = 10000
E = 320000
D = 128
G = 64

NPAD = 10240
NC = 2
NS = 16
NW = NC * NS
C = 128
CH = (E + NW * C - 1) // (NW * C)
CH += CH % 2
EP = NW * CH * C
ZD = NPAD // NS
ZR = 64
BR = 1024
NBLK = NPAD // BR

_sc_mesh = functools.partial(
    plsc.VectorSubcoreMesh, core_axis_name="c", subcore_axis_name="s")

_sc_params = pltpu.CompilerParams()
if "needs_layout_passes" in pltpu.CompilerParams.__dataclass_fields__:
  _sc_params = dataclasses.replace(_sc_params, needs_layout_passes=False)


def _deg_partials(cwdata, zdeg):
  @functools.partial(
      pl.kernel,
      out_type=jax.ShapeDtypeStruct((NC, NPAD), jnp.float32),
      mesh=_sc_mesh(),
      compiler_params=_sc_params,
      scratch_types=[
          pltpu.VMEM((CH, 2, C), jnp.int32),
          pltpu.VMEM_SHARED((NPAD,), jnp.float32),
          pltpu.SemaphoreType.DMA,
      ],
  )
  def k(cw_hbm, z_hbm, out_hbm, cw_v, deg_s, sem):
    core = lax.axis_index("c")
    sub = lax.axis_index("s")
    w = core * NS + sub
    pltpu.sync_copy(z_hbm.at[pl.ds(sub * ZD, ZD)],
                    deg_s.at[pl.ds(sub * ZD, ZD)])
    pltpu.sync_copy(cw_hbm.at[w], cw_v)
    plsc.subcore_barrier()

    cw_f = cw_v.bitcast(jnp.float32)
    pltpu.async_copy(cw_f.at[0, 1], deg_s.at[cw_v.at[0, 0]], sem, add=True)

    @pl.loop(0, CH - 1)
    def _(j):
      pltpu.async_copy(cw_f.at[j + 1, 1], deg_s.at[cw_v.at[j + 1, 0]],
                       sem, add=True)
      pltpu.make_async_copy(cw_f.at[j, 1], deg_s.at[cw_v.at[j, 0]],
                            sem).wait()

    pltpu.make_async_copy(cw_f.at[0, 1], deg_s.at[cw_v.at[0, 0]], sem).wait()
    plsc.subcore_barrier()
    pltpu.sync_copy(deg_s.at[pl.ds(sub * ZD, ZD)],
                    out_hbm.at[core, pl.ds(sub * ZD, ZD)])

  return k(cwdata, zdeg)


def _agg_partials(y, edata):
  @functools.partial(
      pl.kernel,
      out_type=jax.ShapeDtypeStruct((NC, NPAD, D), jnp.float32),
      mesh=_sc_mesh(),
      compiler_params=_sc_params,
      scratch_types=[
          pltpu.VMEM((3, C), jnp.int32),
          pltpu.VMEM((3, C), jnp.int32),
          pltpu.VMEM((2, C), jnp.int32),
          pltpu.VMEM((2, C), jnp.int32),
          pltpu.VMEM((C, D), jnp.float32),
          pltpu.VMEM((C, D), jnp.float32),
          pltpu.VMEM((ZR, D), jnp.float32),
          pltpu.VMEM_SHARED((NPAD, D), jnp.float32),
          pltpu.SemaphoreType.DMA,
          pltpu.SemaphoreType.DMA,
          pltpu.SemaphoreType.DMA,
          pltpu.SemaphoreType.DMA,
          pltpu.SemaphoreType.DMA,
          pltpu.SemaphoreType.DMA,
      ],
  )
  def k(y_hbm, ed_hbm, out_hbm,
        eb0_v, eb1_v, cb0_v, cb1_v, buf0_v, buf1_v, z_v, acc_s,
        es0, es1, gs0, gs1, ss0, ss1):
    core = lax.axis_index("c")
    sub = lax.axis_index("s")
    w = core * NS + sub

    pltpu.async_copy(ed_hbm.at[w, 0], eb0_v, es0)
    pltpu.async_copy(ed_hbm.at[w, 1], eb1_v, es1)

    @pl.loop(0, ZR)
    def _(r):
      for f in range(D // 16):
        z_v[r, pl.ds(f * 16, 16)] = jnp.zeros((16,), jnp.float32)

    for i in range(ZD // ZR):
      pltpu.sync_copy(z_v, acc_s.at[pl.ds(sub * ZD + i * ZR, ZR), :])

    pltpu.make_async_copy(ed_hbm.at[w, 0], eb0_v, es0).wait()
    pltpu.async_copy(y_hbm.at[eb0_v.at[0]], buf0_v, gs0)
    plsc.subcore_barrier()

    ebs = (eb0_v, eb1_v)
    cbs = (cb0_v, cb1_v)
    bufs = (buf0_v, buf1_v)
    esems = (es0, es1)
    gsems = (gs0, gs1)
    ssems = (ss0, ss1)

    one = jnp.full((16,), 1, jnp.int32)

    @pl.loop(0, CH, step=2)
    def _(j):
      for b in (0, 1):
        jj = j + b
        nb = 1 - b
        eb_b, eb_n = ebs[b], ebs[nb]
        cb_b, cb_n = cbs[b], cbs[nb]
        buf_b, buf_n = bufs[b], bufs[nb]

        @pl.when(jj >= 1)
        def _():
          pltpu.make_async_copy(
              buf_n, acc_s.at[cb_n.at[0]], ssems[nb]).wait()

        @pl.when(jj + 1 < CH)
        def _():
          pltpu.make_async_copy(ed_hbm.at[w, jj + 1], eb_n, esems[nb]).wait()
          pltpu.async_copy(y_hbm.at[eb_n.at[0]], buf_n, gsems[nb])

        pltpu.make_async_copy(y_hbm.at[eb_b.at[0]], buf_b, gsems[b]).wait()

        for f in range(D // 16):
          sl = pl.ds(f * 16, 16)
          cb_b[0, sl] = eb_b[1, sl]
          cb_b[1, sl] = eb_b[2, sl]

        @pl.when(jj + 2 < CH)
        def _():
          pltpu.async_copy(ed_hbm.at[w, jj + 2], eb_b, esems[b])

        @plsc.parallel_loop(0, C, unroll=16)
        def _(kk):
          wv = plsc.bitcast(
              plsc.load_gather(cb_b, [one, jnp.full((16,), kk, jnp.int32)]),
              jnp.float32)
          for f in range(D // 16):
            sl = pl.ds(f * 16, 16)
            buf_b[kk, sl] = buf_b[kk, sl] * wv

        pltpu.async_copy(buf_b, acc_s.at[cb_b.at[0]], ssems[b], add=True)

    pltpu.make_async_copy(buf1_v, acc_s.at[cb1_v.at[0]], ss1).wait()
    plsc.subcore_barrier()
    pltpu.sync_copy(acc_s.at[pl.ds(sub * ZD, ZD), :],
                    out_hbm.at[core, pl.ds(sub * ZD, ZD), :])

  return k(y, edata)


def _dinv_of(dp_ref):
  deg = dp_ref[0] + dp_ref[1] + 1.0
  return lax.rsqrt(deg)


def _mm_scale(xp, W, degp3):
  def body(x_ref, w_ref, dp_ref, o_ref):
    dinv = _dinv_of(dp_ref)
    mm = lax.dot_general(x_ref[...].astype(jnp.bfloat16),
                         w_ref[...].astype(jnp.bfloat16),
                         (((1,), (1,)), ((), ())),
                         preferred_element_type=jnp.float32)
    o_ref[...] = mm * dinv

  return pl.pallas_call(
      body,
      grid=(NBLK,),
      in_specs=[
          pl.BlockSpec((BR, D), lambda i: (i, 0)),
          pl.BlockSpec((D, D), lambda i: (0, 0)),
          pl.BlockSpec((2, BR, 1), lambda i: (0, i, 0)),
      ],
      out_specs=pl.BlockSpec((BR, D), lambda i: (i, 0)),
      out_shape=jax.ShapeDtypeStruct((NPAD, D), jnp.float32),
  )(xp, W, degp3)


def _combine_mm(aggp, y, degp3, b2d, W):
  def body(a_ref, y_ref, dp_ref, b_ref, w_ref, o_ref):
    dinv = _dinv_of(dp_ref)
    agg = a_ref[0] + a_ref[1]
    h = jnp.maximum(dinv * (agg + y_ref[...]) + b_ref[...], 0.0)
    mm = lax.dot_general(h.astype(jnp.bfloat16),
                         w_ref[...].astype(jnp.bfloat16),
                         (((1,), (1,)), ((), ())),
                         preferred_element_type=jnp.float32)
    o_ref[...] = mm * dinv

  return pl.pallas_call(
      body,
      grid=(NBLK,),
      in_specs=[
          pl.BlockSpec((2, BR, D), lambda i: (0, i, 0)),
          pl.BlockSpec((BR, D), lambda i: (i, 0)),
          pl.BlockSpec((2, BR, 1), lambda i: (0, i, 0)),
          pl.BlockSpec((1, D), lambda i: (0, 0)),
          pl.BlockSpec((D, D), lambda i: (0, 0)),
      ],
      out_specs=pl.BlockSpec((BR, D), lambda i: (i, 0)),
      out_shape=jax.ShapeDtypeStruct((NPAD, D), jnp.float32),
  )(aggp, y, degp3, b2d, W)


def _final(aggp, y, degp3, b2d, batch3, Wfc, bfc2d):
  def body(a_ref, y_ref, dp_ref, b_ref, bat_ref, wfc_ref, bfc_ref, o_ref,
           ps_ref, cnt_ref):
    i = pl.program_id(0)

    @pl.when(i == 0)
    def _():
      ps_ref[...] = jnp.zeros_like(ps_ref)
      cnt_ref[...] = jnp.zeros_like(cnt_ref)

    dinv = _dinv_of(dp_ref)
    agg = a_ref[0] + a_ref[1]
    h = jnp.maximum(dinv * (agg + y_ref[...]) + b_ref[...], 0.0)
    bat = bat_ref[0]
    oh = (bat == lax.broadcasted_iota(jnp.int32, (1, G), 1)
          ).astype(jnp.float32)
    ps_ref[...] += lax.dot_general(oh, h, (((0,), (0,)), ((), ())),
                                   preferred_element_type=jnp.float32)
    ones = jnp.ones((BR, 1), jnp.float32)
    cnt_ref[...] += lax.dot_general(oh, ones, (((0,), (0,)), ((), ())),
                                    preferred_element_type=jnp.float32)

    @pl.when(i == pl.num_programs(0) - 1)
    def _():
      p = ps_ref[...] / jnp.maximum(cnt_ref[...], 1.0)
      o_ref[...] = lax.dot_general(p, wfc_ref[...], (((1,), (1,)), ((), ())),
                                   preferred_element_type=jnp.float32
                                   ) + bfc_ref[...]

  return pl.pallas_call(
      body,
      grid=(NBLK,),
      in_specs=[
          pl.BlockSpec((2, BR, D), lambda i: (0, i, 0)),
          pl.BlockSpec((BR, D), lambda i: (i, 0)),
          pl.BlockSpec((2, BR, 1), lambda i: (0, i, 0)),
          pl.BlockSpec((1, D), lambda i: (0, 0)),
          pl.BlockSpec((1, BR, 1), lambda i: (i, 0, 0)),
          pl.BlockSpec((D, D), lambda i: (0, 0)),
          pl.BlockSpec((1, D), lambda i: (0, 0)),
      ],
      out_specs=pl.BlockSpec((G, D), lambda i: (0, 0)),
      out_shape=jax.ShapeDtypeStruct((G, D), jnp.float32),
      scratch_shapes=[
          pltpu.VMEM((G, D), jnp.float32),
          pltpu.VMEM((G, 1), jnp.float32),
      ],
  )(aggp, y, degp3, b2d, batch3, Wfc, bfc2d)


@jax.jit
def kernel(x, edge_index, edge_weight, batch, W1, b1, W2, b2, Wfc, bfc):
  xp = jnp.zeros((NPAD, D), jnp.float32).at[:N].set(x)

  pad = EP - E
  padidx = (jnp.arange(pad, dtype=jnp.int32) * 131) % N
  row_p = jnp.concatenate([edge_index[0], padidx]).reshape(NW, CH, C)
  col_p = jnp.concatenate([edge_index[1], padidx]).reshape(NW, CH, C)
  ew_b = lax.bitcast_convert_type(
      jnp.concatenate([edge_weight, jnp.zeros((pad,), jnp.float32)]),
      jnp.int32).reshape(NW, CH, C)
  edata = jnp.stack([row_p, col_p, ew_b], axis=2)
  cwdata = jnp.stack([col_p, ew_b], axis=2)

  batch3 = jnp.concatenate(
      [batch, jnp.full((NPAD - N,), G, jnp.int32)]).reshape(NBLK, BR, 1)

  zdeg = jnp.zeros((NPAD,), jnp.float32)

  degp3 = _deg_partials(cwdata, zdeg).reshape(NC, NPAD, 1)

  y1 = _mm_scale(xp, W1, degp3)
  agg1 = _agg_partials(y1, edata)
  y2 = _combine_mm(agg1, y1, degp3, b1.reshape(1, D), W2)
  agg2 = _agg_partials(y2, edata)
  return _final(agg2, y2, degp3, b2.reshape(1, D), batch3, Wfc,
                bfc.reshape(1, D))
